# baseline (speedup 1.0000x reference)
import functools

import jax
import jax.numpy as jnp
from jax import lax
from jax.experimental import pallas as pl
from jax.experimental.pallas import tpu as pltpu
from jax.experimental.pallas import tpu_sc as plsc


def _dense_body(xn, xg, wn, bn, wg, bg, on, og):
    on[...] = jnp.dot(xn[...], wn[...], preferred_element_type=jnp.float32) + bn[...]
    og[...] = jnp.dot(xg[...], wg[...], preferred_element_type=jnp.float32) + bg[...]


def _dense_tokens(X_num, X_geo, W_num, b_num, W_geo, b_geo):
    B, NUM = X_num.shape
    NGEO = X_geo.shape[1]
    D = W_num.shape[1]
    bs = 2048
    out = pl.pallas_call(
        _dense_body,
        grid=(B // bs,),
        in_specs=[
            pl.BlockSpec((bs, NUM), lambda i: (i, 0)),
            pl.BlockSpec((bs, NGEO), lambda i: (i, 0)),
            pl.BlockSpec((NUM, D), lambda i: (0, 0)),
            pl.BlockSpec((1, D), lambda i: (0, 0)),
            pl.BlockSpec((NGEO, D), lambda i: (0, 0)),
            pl.BlockSpec((1, D), lambda i: (0, 0)),
        ],
        out_specs=[
            pl.BlockSpec((bs, D), lambda i: (i, 0)),
            pl.BlockSpec((bs, D), lambda i: (i, 0)),
        ],
        out_shape=[
            jax.ShapeDtypeStruct((B, D), jnp.float32),
            jax.ShapeDtypeStruct((B, D), jnp.float32),
        ],
    )(X_num, X_geo, W_num, b_num.reshape(1, D), W_geo, b_geo.reshape(1, D))
    return out


@functools.lru_cache(maxsize=None)
def _make_sc_tokenizer(B, NCAT, VOCAB, D, NC, NS, L, chunks):
    NT = NCAT + 2
    NW = NC * NS
    RPW = B // NW
    G = RPW // 128
    assert RPW % 128 == 0 and sum(chunks) == NCAT

    mesh = plsc.VectorSubcoreMesh(core_axis_name="c", subcore_axis_name="s")

    @functools.partial(
        pl.kernel,
        out_type=jax.ShapeDtypeStruct((B, NT * D), jnp.float32),
        mesh=mesh,
        compiler_params=pltpu.CompilerParams(use_tc_tiling_on_sc=False),
        scratch_types=[
            pltpu.VMEM((RPW,), jnp.int32),
            pltpu.VMEM((RPW, D), jnp.float32),
            pltpu.VMEM((RPW, D), jnp.float32),
            pltpu.SemaphoreType.DMA,
        ],
    )
    def sc_tok(*args):
        tabs = args[:len(chunks)]
        (xcatT, numt, geot, out2, idx, rows, dstage, sem) = args[len(chunks):]
        wid = lax.axis_index("s") * NC + lax.axis_index("c")
        b0 = wid * RPW

        pltpu.sync_copy(numt.at[pl.ds(b0, RPW), :], dstage)
        pltpu.sync_copy(dstage, out2.at[pl.ds(b0, RPW), pl.ds(0, D)])
        pltpu.sync_copy(geot.at[pl.ds(b0, RPW), :], dstage)
        pltpu.sync_copy(dstage, out2.at[pl.ds(b0, RPW), pl.ds((NT - 1) * D, D)])

        base = 0
        for k, nk in enumerate(chunks):
            tab = tabs[k]

            def field_body(f, carry, tab=tab, base=base):
                pltpu.sync_copy(xcatT.at[base + f, pl.ds(b0, RPW)], idx)
                gathers = [
                    pltpu.async_copy(tab.at[f].at[idx.at[pl.ds(g * 128, 128)]],
                                     rows.at[pl.ds(g * 128, 128), :], sem)
                    for g in range(G)
                ]
                for cp in gathers:
                    cp.wait()
                pltpu.sync_copy(
                    rows,
                    out2.at[pl.ds(b0, RPW), pl.dslice((base + f + 1) * D, D)])
                return carry

            lax.fori_loop(0, nk, field_body, 0)
            base += nk

    return sc_tok


def kernel(X_num, X_cat, X_geo, W_num, b_num, tables, W_geo, b_geo):
    B = X_num.shape[0]
    NCAT, VOCAB, D = tables.shape
    try:
        info = plsc.get_sparse_core_info()
        NC, NS, L = info.num_cores, info.num_subcores, info.num_lanes
    except Exception:
        NC, NS, L = 2, 16, 16

    numt, geot = _dense_tokens(X_num, X_geo, W_num, b_num, W_geo, b_geo)
    nch = 6
    sizes = [NCAT // nch + (1 if i < NCAT % nch else 0) for i in range(nch)]
    sizes = [s for s in sizes if s]
    sc_tok = _make_sc_tokenizer(B, NCAT, VOCAB, D, NC, NS, L, tuple(sizes))
    tabs, s0 = [], 0
    for s in sizes:
        tabs.append(lax.slice_in_dim(tables, s0, s0 + s, axis=0))
        s0 += s
    out2 = sc_tok(*tabs, X_cat.T, numt, geot)
    return out2.reshape(B, NCAT + 2, D)


# speedup vs baseline: 1.4553x; 1.4553x over previous
import functools

import jax
import jax.numpy as jnp
from jax import lax
from jax.experimental import pallas as pl
from jax.experimental.pallas import tpu as pltpu
from jax.experimental.pallas import tpu_sc as plsc


def _dense_body(xn, xg, wn, bn, wg, bg, on, og):
    on[...] = jnp.dot(xn[...], wn[...], preferred_element_type=jnp.float32) + bn[...]
    og[...] = jnp.dot(xg[...], wg[...], preferred_element_type=jnp.float32) + bg[...]


def _dense_tokens(X_num, X_geo, W_num, b_num, W_geo, b_geo):
    B, NUM = X_num.shape
    NGEO = X_geo.shape[1]
    D = W_num.shape[1]
    bs = 2048
    out = pl.pallas_call(
        _dense_body,
        grid=(B // bs,),
        in_specs=[
            pl.BlockSpec((bs, NUM), lambda i: (i, 0)),
            pl.BlockSpec((bs, NGEO), lambda i: (i, 0)),
            pl.BlockSpec((NUM, D), lambda i: (0, 0)),
            pl.BlockSpec((1, D), lambda i: (0, 0)),
            pl.BlockSpec((NGEO, D), lambda i: (0, 0)),
            pl.BlockSpec((1, D), lambda i: (0, 0)),
        ],
        out_specs=[
            pl.BlockSpec((bs, D), lambda i: (i, 0)),
            pl.BlockSpec((bs, D), lambda i: (i, 0)),
        ],
        out_shape=[
            jax.ShapeDtypeStruct((B, D), jnp.float32),
            jax.ShapeDtypeStruct((B, D), jnp.float32),
        ],
    )(X_num, X_geo, W_num, b_num.reshape(1, D), W_geo, b_geo.reshape(1, D))
    return out


@functools.lru_cache(maxsize=None)
def _make_sc_tokenizer(B, NCAT, VOCAB, D, NC, NS, L, chunks):
    NT = NCAT + 2
    NW = NC * NS
    RPW = B // NW
    G = RPW // 128
    assert RPW % 128 == 0 and sum(chunks) == NCAT

    mesh = plsc.VectorSubcoreMesh(core_axis_name="c", subcore_axis_name="s")

    @functools.partial(
        pl.kernel,
        out_type=jax.ShapeDtypeStruct((B, NT * D), jnp.float32),
        mesh=mesh,
        compiler_params=pltpu.CompilerParams(use_tc_tiling_on_sc=False),
        scratch_types=[
            pltpu.VMEM((RPW,), jnp.int32),
            pltpu.VMEM((RPW, D), jnp.float32),
            pltpu.VMEM((RPW, D), jnp.float32),
            pltpu.SemaphoreType.DMA,
        ],
    )
    def sc_tok(*args):
        tabs = args[:len(chunks)]
        (xcatT, numt, geot, out2, idx, rows, dstage, sem) = args[len(chunks):]
        wid = lax.axis_index("s") * NC + lax.axis_index("c")
        b0 = wid * RPW

        pltpu.sync_copy(numt.at[pl.ds(b0, RPW), :], dstage)
        pltpu.sync_copy(dstage, out2.at[pl.ds(b0, RPW), pl.ds(0, D)])
        pltpu.sync_copy(geot.at[pl.ds(b0, RPW), :], dstage)
        pltpu.sync_copy(dstage, out2.at[pl.ds(b0, RPW), pl.ds((NT - 1) * D, D)])

        base = 0
        for k, nk in enumerate(chunks):
            tab = tabs[k]

            def field_body(f, carry, tab=tab, base=base):
                pltpu.sync_copy(xcatT.at[base + f, pl.ds(b0, RPW)], idx)
                gathers = [
                    pltpu.async_copy(tab.at[f].at[idx.at[pl.ds(g * 128, 128)]],
                                     rows.at[pl.ds(g * 128, 128), :], sem)
                    for g in range(G)
                ]
                for cp in gathers:
                    cp.wait()
                pltpu.sync_copy(
                    rows,
                    out2.at[pl.ds(b0, RPW), pl.dslice((base + f + 1) * D, D)])
                return carry

            lax.fori_loop(0, nk, field_body, 0)
            base += nk

    return sc_tok


def kernel(X_num, X_cat, X_geo, W_num, b_num, tables, W_geo, b_geo):
    B = X_num.shape[0]
    NCAT, VOCAB, D = tables.shape
    try:
        info = plsc.get_sparse_core_info()
        NC, NS, L = info.num_cores, info.num_subcores, info.num_lanes
    except Exception:
        NC, NS, L = 2, 16, 16

    numt, geot = _dense_tokens(X_num, X_geo, W_num, b_num, W_geo, b_geo)
    sc_tok = _make_sc_tokenizer(B, NCAT, VOCAB, D, NC, NS, L, (NCAT,))
    out2 = sc_tok(tables, X_cat.T, numt, geot)
    return out2.reshape(B, NCAT + 2, D)


# speedup vs baseline: 1.4762x; 1.0144x over previous
import functools

import jax
import jax.numpy as jnp
from jax import lax
from jax.experimental import pallas as pl
from jax.experimental.pallas import tpu as pltpu
from jax.experimental.pallas import tpu_sc as plsc


def _dense_body(xn, xg, wn, bn, wg, bg, on, og):
    on[...] = jnp.dot(xn[...], wn[...], preferred_element_type=jnp.float32) + bn[...]
    og[...] = jnp.dot(xg[...], wg[...], preferred_element_type=jnp.float32) + bg[...]


def _dense_tokens(X_num, X_geo, W_num, b_num, W_geo, b_geo):
    B, NUM = X_num.shape
    NGEO = X_geo.shape[1]
    D = W_num.shape[1]
    bs = 2048
    out = pl.pallas_call(
        _dense_body,
        grid=(B // bs,),
        in_specs=[
            pl.BlockSpec((bs, NUM), lambda i: (i, 0)),
            pl.BlockSpec((bs, NGEO), lambda i: (i, 0)),
            pl.BlockSpec((NUM, D), lambda i: (0, 0)),
            pl.BlockSpec((1, D), lambda i: (0, 0)),
            pl.BlockSpec((NGEO, D), lambda i: (0, 0)),
            pl.BlockSpec((1, D), lambda i: (0, 0)),
        ],
        out_specs=[
            pl.BlockSpec((bs, D), lambda i: (i, 0)),
            pl.BlockSpec((bs, D), lambda i: (i, 0)),
        ],
        out_shape=[
            jax.ShapeDtypeStruct((B, D), jnp.float32),
            jax.ShapeDtypeStruct((B, D), jnp.float32),
        ],
    )(X_num, X_geo, W_num, b_num.reshape(1, D), W_geo, b_geo.reshape(1, D))
    return out


@functools.lru_cache(maxsize=None)
def _make_sc_tokenizer(B, NCAT, VOCAB, D, NC, NS, L, chunks):
    NT = NCAT + 2
    NW = NC * NS
    RPW = B // NW
    G = RPW // 128
    assert RPW % 128 == 0 and sum(chunks) == NCAT

    mesh = plsc.VectorSubcoreMesh(core_axis_name="c", subcore_axis_name="s")

    @functools.partial(
        pl.kernel,
        out_type=jax.ShapeDtypeStruct((B, NT * D), jnp.float32),
        mesh=mesh,
        compiler_params=pltpu.CompilerParams(use_tc_tiling_on_sc=False),
        scratch_types=[
            pltpu.VMEM((2, RPW), jnp.int32),
            pltpu.VMEM((2, RPW, D), jnp.float32),
            pltpu.VMEM((RPW, D), jnp.float32),
            pltpu.SemaphoreType.DMA,
            pltpu.SemaphoreType.DMA,
            pltpu.SemaphoreType.DMA,
        ],
    )
    def sc_tok(*args):
        tabs = args[:len(chunks)]
        (xcatT, numt, geot, out2,
         idx, rows, dstage, gsem, wsem0, wsem1) = args[len(chunks):]
        assert len(chunks) == 1
        tab = tabs[0]
        wsems = (wsem0, wsem1)
        wid = lax.axis_index("s") * NC + lax.axis_index("c")
        b0 = wid * RPW

        pltpu.sync_copy(numt.at[pl.ds(b0, RPW), :], dstage)
        pltpu.sync_copy(dstage, out2.at[pl.ds(b0, RPW), pl.ds(0, D)])
        pltpu.sync_copy(geot.at[pl.ds(b0, RPW), :], dstage)
        pltpu.sync_copy(dstage, out2.at[pl.ds(b0, RPW), pl.ds((NT - 1) * D, D)])

        pltpu.sync_copy(xcatT.at[0, pl.ds(b0, RPW)], idx.at[0])
        writes = [None, None]
        for f in range(NCAT):
            p = f % 2
            if writes[p] is not None:
                writes[p].wait()
            gathers = [
                pltpu.async_copy(
                    tab.at[f].at[idx.at[p].at[pl.ds(g * 128, 128)]],
                    rows.at[p].at[pl.ds(g * 128, 128), :], gsem)
                for g in range(G)
            ]
            if f + 1 < NCAT:
                pltpu.sync_copy(xcatT.at[f + 1, pl.ds(b0, RPW)],
                                idx.at[1 - p])
            for cp in gathers:
                cp.wait()
            writes[p] = pltpu.async_copy(
                rows.at[p],
                out2.at[pl.ds(b0, RPW), pl.ds((f + 1) * D, D)], wsems[p])
        for wr in writes:
            wr.wait()

    return sc_tok


def kernel(X_num, X_cat, X_geo, W_num, b_num, tables, W_geo, b_geo):
    B = X_num.shape[0]
    NCAT, VOCAB, D = tables.shape
    try:
        info = plsc.get_sparse_core_info()
        NC, NS, L = info.num_cores, info.num_subcores, info.num_lanes
    except Exception:
        NC, NS, L = 2, 16, 16

    numt, geot = _dense_tokens(X_num, X_geo, W_num, b_num, W_geo, b_geo)
    sc_tok = _make_sc_tokenizer(B, NCAT, VOCAB, D, NC, NS, L, (NCAT,))
    out2 = sc_tok(tables, X_cat.T, numt, geot)
    return out2.reshape(B, NCAT + 2, D)
